# Initial kernel scaffold; baseline (speedup 1.0000x reference)
import jax
import jax.numpy as jnp
from jax.experimental import pallas as pl


def kernel(x, edge_index, batch, W1, b1, Wm1, bm1, W2, b2, W3, b3, Wl, bl):
    raise NotImplementedError("write your pallas kernel here")


# speedup vs baseline: 27.0439x; 27.0439x over previous
import functools

import jax
import jax.numpy as jnp
from jax import lax
from jax.experimental import pallas as pl
from jax.experimental.pallas import tpu as pltpu
from jax.experimental.pallas import tpu_sc as plsc

NC = 2
NS = 16
NW = NC * NS
C = 128


def _mesh():
    return plsc.VectorSubcoreMesh(
        core_axis_name="c", subcore_axis_name="s", num_cores=NC, num_subcores=NS
    )


_SC_PARAMS = pltpu.CompilerParams(use_tc_tiling_on_sc=False)


def _deg_call(dst_t, n_acc, f):
    chunks = dst_t.shape[1]
    npt = n_acc // NS

    @functools.partial(
        pl.kernel,
        out_type=jax.ShapeDtypeStruct((NC, n_acc, f), jnp.float32),
        mesh=_mesh(),
        compiler_params=_SC_PARAMS,
        scratch_types=[
            pltpu.VMEM((chunks, C), jnp.int32),
            pltpu.VMEM((C, f), jnp.float32),
            pltpu.VMEM((C, f), jnp.float32),
            pltpu.VMEM_SHARED((n_acc, f), jnp.float32),
        ],
    )
    def deg_kernel(dst_hbm, out_hbm, dst_v, ones_v, zero_v, acc_sh):
        cid = lax.axis_index("c")
        sid = lax.axis_index("s")
        wid = sid * NC + cid

        pltpu.sync_copy(dst_hbm.at[wid], dst_v)

        def fill(i, carry):
            ones_v[i, :] = jnp.full((f,), 1.0, jnp.float32)
            zero_v[i, :] = jnp.zeros((f,), jnp.float32)
            return carry
        lax.fori_loop(0, C, fill, 0)

        base = sid * npt
        for off in range(0, npt, C):
            w = min(C, npt - off)
            pltpu.sync_copy(zero_v.at[pl.ds(0, w)], acc_sh.at[pl.ds(base + off, w)])
        plsc.subcore_barrier()

        def chunk(j, carry):
            pltpu.sync_copy(ones_v, acc_sh.at[dst_v.at[j]], add=True)
            return carry
        lax.fori_loop(0, chunks, chunk, 0)
        plsc.subcore_barrier()

        pltpu.sync_copy(acc_sh.at[pl.ds(base, npt)],
                        out_hbm.at[cid, pl.ds(base, npt)])

    return deg_kernel(dst_t)


def _prop_call(g, src_t, dst_t, n_acc):
    f = g.shape[1]
    chunks = src_t.shape[1]
    npt = n_acc // NS

    @functools.partial(
        pl.kernel,
        out_type=jax.ShapeDtypeStruct((NC, n_acc, f), jnp.float32),
        mesh=_mesh(),
        compiler_params=_SC_PARAMS,
        scratch_types=[
            pltpu.VMEM((chunks, C), jnp.int32),
            pltpu.VMEM((chunks, C), jnp.int32),
            pltpu.VMEM((C, f), jnp.float32),
            pltpu.VMEM((C, f), jnp.float32),
            pltpu.VMEM_SHARED((n_acc, f), jnp.float32),
            pltpu.SemaphoreType.DMA,
        ],
    )
    def prop_kernel(g_hbm, src_hbm, dst_hbm, out_hbm,
                    src_v, dst_v, rows_v, zero_v, acc_sh, sem):
        cid = lax.axis_index("c")
        sid = lax.axis_index("s")
        wid = sid * NC + cid

        pltpu.sync_copy(src_hbm.at[wid], src_v)
        pltpu.sync_copy(dst_hbm.at[wid], dst_v)

        def fill(i, carry):
            zero_v[i, :] = jnp.zeros((f,), jnp.float32)
            return carry
        lax.fori_loop(0, C, fill, 0)

        base = sid * npt
        for off in range(0, npt, C):
            w = min(C, npt - off)
            pltpu.sync_copy(zero_v.at[pl.ds(0, w)], acc_sh.at[pl.ds(base + off, w)])
        plsc.subcore_barrier()

        def chunk(j, carry):
            pltpu.async_copy(g_hbm.at[src_v.at[j]], rows_v, sem).wait()
            pltpu.sync_copy(rows_v, acc_sh.at[dst_v.at[j]], add=True)
            return carry
        lax.fori_loop(0, chunks, chunk, 0)
        plsc.subcore_barrier()

        pltpu.sync_copy(acc_sh.at[pl.ds(base, npt)],
                        out_hbm.at[cid, pl.ds(base, npt)])

    return prop_kernel(g, src_t, dst_t)


def _k1_body(x_ref, w1_ref, degp_ref, g1_ref, dinv_ref):
    n = x_ref.shape[0]
    n_acc = degp_ref.shape[1]
    f = w1_ref.shape[1]
    deg = degp_ref[0] + degp_ref[1] + 1.0
    dinv = lax.rsqrt(deg)
    dinv_ref[...] = dinv
    t1 = jnp.dot(x_ref[...], w1_ref[...], preferred_element_type=jnp.float32)
    t1p = jnp.concatenate([t1, jnp.zeros((n_acc - n, f), jnp.float32)], axis=0)
    g1_ref[...] = dinv * t1p


def _k2_body(acc_ref, g_ref, dinv_ref, b1_ref, wm1_ref, bm1_ref, w2_ref, g2_ref):
    s = acc_ref[0] + acc_ref[1] + g_ref[...]
    h1 = jnp.maximum(dinv_ref[...] * s + b1_ref[...], 0.0)
    h2 = jnp.maximum(
        jnp.dot(h1, wm1_ref[...], preferred_element_type=jnp.float32) + bm1_ref[...],
        0.0)
    g2_ref[...] = dinv_ref[...] * jnp.dot(
        h2, w2_ref[...], preferred_element_type=jnp.float32)


def _k3_body(acc_ref, g_ref, dinv_ref, b2_ref, w3_ref, g3_ref):
    s = acc_ref[0] + acc_ref[1] + g_ref[...]
    h3 = jnp.maximum(dinv_ref[...] * s + b2_ref[...], 0.0)
    g3_ref[...] = dinv_ref[...] * jnp.dot(
        h3, w3_ref[...], preferred_element_type=jnp.float32)


def _k4_body(acc_ref, g_ref, dinv_ref, b3_ref, batch_ref, wl_ref, bl_ref, out_ref):
    n, g_out = batch_ref.shape[0], out_ref.shape[0]
    f = g_ref.shape[1]
    s = acc_ref[0] + acc_ref[1] + g_ref[...]
    h = jnp.maximum(dinv_ref[...] * s + b3_ref[...], 0.0)[:n]
    onehot = (batch_ref[...] == lax.broadcasted_iota(jnp.int32, (1, g_out), 1)
              ).astype(jnp.float32)
    aug = jnp.concatenate([h, jnp.ones((n, 1), jnp.float32)], axis=1)
    sums = lax.dot_general(onehot, aug, (((0,), (0,)), ((), ())),
                           preferred_element_type=jnp.float32)
    pooled = sums[:, :f] / jnp.maximum(sums[:, f:f + 1], 1.0)
    out_ref[...] = jnp.dot(pooled, wl_ref[...],
                           preferred_element_type=jnp.float32) + bl_ref[...]


def _tc(body, out_shapes):
    return pl.pallas_call(body, out_shape=out_shapes)


def kernel(x, edge_index, batch, W1, b1, Wm1, bm1, W2, b2, W3, b3, Wl, bl):
    n, f_in = x.shape
    f = W1.shape[1]
    e = edge_index.shape[1]
    num_graphs = 32

    n_acc = ((n + 1 + 8 * NS - 1) // (8 * NS)) * (8 * NS)
    chunks = -(-e // (NW * C))
    if chunks % 2:
        chunks += 1
    ept_p = chunks * C

    pad = NW * ept_p - e
    dummy = jnp.full((pad,), n, jnp.int32)
    src_t = jnp.concatenate([edge_index[0], dummy]).reshape(NW, chunks, C)
    dst_t = jnp.concatenate([edge_index[1], dummy]).reshape(NW, chunks, C)

    degp = _deg_call(dst_t, n_acc, f)

    sd32 = jax.ShapeDtypeStruct((n_acc, f), jnp.float32)
    g1, dinv = _tc(_k1_body, [sd32, sd32])(x, W1, degp)

    acc1 = _prop_call(g1, src_t, dst_t, n_acc)
    g2 = _tc(_k2_body, sd32)(acc1, g1, dinv, b1.reshape(1, f), Wm1,
                             bm1.reshape(1, f), W2)
    acc2 = _prop_call(g2, src_t, dst_t, n_acc)
    g3 = _tc(_k3_body, sd32)(acc2, g2, dinv, b2.reshape(1, f), W3)
    acc3 = _prop_call(g3, src_t, dst_t, n_acc)
    out = _tc(_k4_body, jax.ShapeDtypeStruct((num_graphs, 1), jnp.float32))(
        acc3, g3, dinv, b3.reshape(1, f), batch.reshape(n, 1), Wl,
        bl.reshape(1, 1))
    return out


# speedup vs baseline: 37.7354x; 1.3953x over previous
import functools

import jax
import jax.numpy as jnp
from jax import lax
from jax.experimental import pallas as pl
from jax.experimental.pallas import tpu as pltpu
from jax.experimental.pallas import tpu_sc as plsc

NC = 2
NS = 16
NW = NC * NS
C = 128
NBUF = 8


def _mesh():
    return plsc.VectorSubcoreMesh(
        core_axis_name="c", subcore_axis_name="s", num_cores=NC, num_subcores=NS
    )


_SC_PARAMS = pltpu.CompilerParams(use_tc_tiling_on_sc=False)


def _deg_call(dst_t, n_acc, f):
    chunks = dst_t.shape[1]
    npt = n_acc // NS

    @functools.partial(
        pl.kernel,
        out_type=jax.ShapeDtypeStruct((NC, n_acc, f), jnp.float32),
        mesh=_mesh(),
        compiler_params=_SC_PARAMS,
        scratch_types=[
            pltpu.VMEM((chunks, C), jnp.int32),
            pltpu.VMEM((C, f), jnp.float32),
            pltpu.VMEM((C, f), jnp.float32),
            pltpu.VMEM_SHARED((n_acc, f), jnp.float32),
        ],
    )
    def deg_kernel(dst_hbm, out_hbm, dst_v, ones_v, zero_v, acc_sh):
        cid = lax.axis_index("c")
        sid = lax.axis_index("s")
        wid = sid * NC + cid

        pltpu.sync_copy(dst_hbm.at[wid], dst_v)

        def fill(i, carry):
            ones_v[i, :] = jnp.full((f,), 1.0, jnp.float32)
            zero_v[i, :] = jnp.zeros((f,), jnp.float32)
            return carry
        lax.fori_loop(0, C, fill, 0)

        base = sid * npt
        for off in range(0, npt, C):
            w = min(C, npt - off)
            pltpu.sync_copy(zero_v.at[pl.ds(0, w)], acc_sh.at[pl.ds(base + off, w)])
        plsc.subcore_barrier()

        def chunk(j, carry):
            pltpu.sync_copy(ones_v, acc_sh.at[dst_v.at[j]], add=True)
            return carry
        lax.fori_loop(0, chunks, chunk, 0)
        plsc.subcore_barrier()

        pltpu.sync_copy(acc_sh.at[pl.ds(base, npt)],
                        out_hbm.at[cid, pl.ds(base, npt)])

    return deg_kernel(dst_t)


def _prop_call(g, src_t, dst_t, n_acc):
    f = g.shape[1]
    chunks = src_t.shape[1]
    npt = n_acc // NS

    @functools.partial(
        pl.kernel,
        out_type=jax.ShapeDtypeStruct((NC, n_acc, f), jnp.float32),
        mesh=_mesh(),
        compiler_params=_SC_PARAMS,
        scratch_types=[
            pltpu.VMEM((chunks, C), jnp.int32),
            pltpu.VMEM((chunks, C), jnp.int32),
            pltpu.VMEM((NBUF, C, f), jnp.float32),
            pltpu.VMEM((C, f), jnp.float32),
            pltpu.VMEM_SHARED((n_acc, f), jnp.float32),
            pltpu.SemaphoreType.DMA((NBUF,)),
            pltpu.SemaphoreType.DMA((NBUF,)),
        ],
    )
    def prop_kernel(g_hbm, src_hbm, dst_hbm, out_hbm,
                    src_v, dst_v, rows_v, zero_v, acc_sh, sem_g, sem_s):
        cid = lax.axis_index("c")
        sid = lax.axis_index("s")
        wid = sid * NC + cid

        pltpu.sync_copy(src_hbm.at[wid], src_v)
        pltpu.sync_copy(dst_hbm.at[wid], dst_v)

        def fill(i, carry):
            zero_v[i, :] = jnp.zeros((f,), jnp.float32)
            return carry
        lax.fori_loop(0, C, fill, 0)

        base = sid * npt
        for off in range(0, npt, C):
            w = min(C, npt - off)
            pltpu.sync_copy(zero_v.at[pl.ds(0, w)], acc_sh.at[pl.ds(base + off, w)])
        plsc.subcore_barrier()

        D = NBUF - 1
        for d in range(D):
            pltpu.async_copy(g_hbm.at[src_v.at[d]], rows_v.at[d], sem_g.at[d])

        def chunk(j, carry):
            b = j % NBUF
            pltpu.make_async_copy(g_hbm.at[src_v.at[j]], rows_v.at[b],
                                  sem_g.at[b]).wait()
            pltpu.make_async_copy(rows_v.at[b], acc_sh.at[dst_v.at[j]],
                                  sem_s.at[b]).start(add=True)

            @pl.when(j >= 1)
            def _():
                bp = (j - 1) % NBUF
                pltpu.make_async_copy(rows_v.at[bp], acc_sh.at[dst_v.at[j - 1]],
                                      sem_s.at[bp]).wait()

            @pl.when(j + D < chunks)
            def _():
                bn = (j + D) % NBUF
                pltpu.async_copy(g_hbm.at[src_v.at[j + D]], rows_v.at[bn],
                                 sem_g.at[bn])
            return carry
        lax.fori_loop(0, chunks, chunk, 0)
        b_last = (chunks - 1) % NBUF
        pltpu.make_async_copy(rows_v.at[b_last], acc_sh.at[dst_v.at[chunks - 1]],
                              sem_s.at[b_last]).wait()
        plsc.subcore_barrier()

        pltpu.sync_copy(acc_sh.at[pl.ds(base, npt)],
                        out_hbm.at[cid, pl.ds(base, npt)])

    return prop_kernel(g, src_t, dst_t)


def _k1_body(x_ref, w1_ref, degp_ref, g1_ref, dinv_ref):
    n = x_ref.shape[0]
    n_acc = degp_ref.shape[1]
    f = w1_ref.shape[1]
    deg = degp_ref[0] + degp_ref[1] + 1.0
    dinv = lax.rsqrt(deg)
    dinv_ref[...] = dinv
    t1 = jnp.dot(x_ref[...], w1_ref[...], preferred_element_type=jnp.float32)
    t1p = jnp.concatenate([t1, jnp.zeros((n_acc - n, f), jnp.float32)], axis=0)
    g1_ref[...] = dinv * t1p


def _k2_body(acc_ref, g_ref, dinv_ref, b1_ref, wm1_ref, bm1_ref, w2_ref, g2_ref):
    s = acc_ref[0] + acc_ref[1] + g_ref[...]
    h1 = jnp.maximum(dinv_ref[...] * s + b1_ref[...], 0.0)
    h2 = jnp.maximum(
        jnp.dot(h1, wm1_ref[...], preferred_element_type=jnp.float32) + bm1_ref[...],
        0.0)
    g2_ref[...] = dinv_ref[...] * jnp.dot(
        h2, w2_ref[...], preferred_element_type=jnp.float32)


def _k3_body(acc_ref, g_ref, dinv_ref, b2_ref, w3_ref, g3_ref):
    s = acc_ref[0] + acc_ref[1] + g_ref[...]
    h3 = jnp.maximum(dinv_ref[...] * s + b2_ref[...], 0.0)
    g3_ref[...] = dinv_ref[...] * jnp.dot(
        h3, w3_ref[...], preferred_element_type=jnp.float32)


def _k4_body(acc_ref, g_ref, dinv_ref, b3_ref, batch_ref, wl_ref, bl_ref, out_ref):
    n, g_out = batch_ref.shape[0], out_ref.shape[0]
    f = g_ref.shape[1]
    s = acc_ref[0] + acc_ref[1] + g_ref[...]
    h = jnp.maximum(dinv_ref[...] * s + b3_ref[...], 0.0)[:n]
    onehot = (batch_ref[...] == lax.broadcasted_iota(jnp.int32, (1, g_out), 1)
              ).astype(jnp.float32)
    aug = jnp.concatenate([h, jnp.ones((n, 1), jnp.float32)], axis=1)
    sums = lax.dot_general(onehot, aug, (((0,), (0,)), ((), ())),
                           preferred_element_type=jnp.float32)
    pooled = sums[:, :f] / jnp.maximum(sums[:, f:f + 1], 1.0)
    out_ref[...] = jnp.dot(pooled, wl_ref[...],
                           preferred_element_type=jnp.float32) + bl_ref[...]


def _tc(body, out_shapes):
    return pl.pallas_call(body, out_shape=out_shapes)


def kernel(x, edge_index, batch, W1, b1, Wm1, bm1, W2, b2, W3, b3, Wl, bl):
    n, f_in = x.shape
    f = W1.shape[1]
    e = edge_index.shape[1]
    num_graphs = 32

    n_acc = ((n + 1 + 8 * NS - 1) // (8 * NS)) * (8 * NS)
    chunks = -(-e // (NW * C))
    if chunks % 2:
        chunks += 1
    ept_p = chunks * C

    pad = NW * ept_p - e
    dummy = jnp.full((pad,), n, jnp.int32)
    src_t = jnp.concatenate([edge_index[0], dummy]).reshape(NW, chunks, C)
    dst_t = jnp.concatenate([edge_index[1], dummy]).reshape(NW, chunks, C)

    degp = _deg_call(dst_t, n_acc, f)

    sd32 = jax.ShapeDtypeStruct((n_acc, f), jnp.float32)
    g1, dinv = _tc(_k1_body, [sd32, sd32])(x, W1, degp)

    acc1 = _prop_call(g1, src_t, dst_t, n_acc)
    g2 = _tc(_k2_body, sd32)(acc1, g1, dinv, b1.reshape(1, f), Wm1,
                             bm1.reshape(1, f), W2)
    acc2 = _prop_call(g2, src_t, dst_t, n_acc)
    g3 = _tc(_k3_body, sd32)(acc2, g2, dinv, b2.reshape(1, f), W3)
    acc3 = _prop_call(g3, src_t, dst_t, n_acc)
    out = _tc(_k4_body, jax.ShapeDtypeStruct((num_graphs, 1), jnp.float32))(
        acc3, g3, dinv, b3.reshape(1, f), batch.reshape(n, 1), Wl,
        bl.reshape(1, 1))
    return out


# speedup vs baseline: 54.3474x; 1.4402x over previous
import functools

import jax
import jax.numpy as jnp
from jax import lax
from jax.experimental import pallas as pl
from jax.experimental.pallas import tpu as pltpu
from jax.experimental.pallas import tpu_sc as plsc

NC = 2
NS = 16
NW = NC * NS
C = 128
NBUF = 8


def _mesh():
    return plsc.VectorSubcoreMesh(
        core_axis_name="c", subcore_axis_name="s", num_cores=NC, num_subcores=NS
    )


_SC_PARAMS = pltpu.CompilerParams(use_tc_tiling_on_sc=False)


def _deg_call(dst_t, n_acc, f):
    chunks = dst_t.shape[1]
    npt = n_acc // NS

    @functools.partial(
        pl.kernel,
        out_type=jax.ShapeDtypeStruct((NC, n_acc, f), jnp.float32),
        mesh=_mesh(),
        compiler_params=_SC_PARAMS,
        scratch_types=[
            pltpu.VMEM((chunks, C), jnp.int32),
            pltpu.VMEM((C, f), jnp.float32),
            pltpu.VMEM((C, f), jnp.float32),
            pltpu.VMEM_SHARED((n_acc, f), jnp.float32),
        ],
    )
    def deg_kernel(dst_hbm, out_hbm, dst_v, ones_v, zero_v, acc_sh):
        cid = lax.axis_index("c")
        sid = lax.axis_index("s")
        wid = sid * NC + cid

        pltpu.sync_copy(dst_hbm.at[wid], dst_v)

        def fill(i, carry):
            ones_v[i, :] = jnp.full((f,), 1.0, jnp.float32)
            zero_v[i, :] = jnp.zeros((f,), jnp.float32)
            return carry
        lax.fori_loop(0, C, fill, 0)

        base = sid * npt
        for off in range(0, npt, C):
            w = min(C, npt - off)
            pltpu.sync_copy(zero_v.at[pl.ds(0, w)], acc_sh.at[pl.ds(base + off, w)])
        plsc.subcore_barrier()

        def chunk(j, carry):
            pltpu.sync_copy(ones_v, acc_sh.at[dst_v.at[j]], add=True)
            return carry
        lax.fori_loop(0, chunks, chunk, 0)
        plsc.subcore_barrier()

        pltpu.sync_copy(acc_sh.at[pl.ds(base, npt)],
                        out_hbm.at[cid, pl.ds(base, npt)])

    return deg_kernel(dst_t)


def _prop_call(g, src_t, dst_t, n_acc):
    f = g.shape[1]
    chunks = src_t.shape[1]
    npt = n_acc // NS

    @functools.partial(
        pl.kernel,
        out_type=jax.ShapeDtypeStruct((NC, n_acc, f), jnp.float32),
        mesh=_mesh(),
        compiler_params=_SC_PARAMS,
        scratch_types=[
            pltpu.VMEM((chunks, C), jnp.int32),
            pltpu.VMEM((chunks, C), jnp.int32),
            pltpu.VMEM((NBUF, C, f), jnp.float32),
            pltpu.VMEM((C, f), jnp.float32),
            pltpu.VMEM_SHARED((n_acc, f), jnp.float32),
            pltpu.VMEM_SHARED((n_acc, f), jnp.float32),
            pltpu.SemaphoreType.DMA((NBUF,)),
            pltpu.SemaphoreType.DMA((NBUF,)),
        ],
    )
    def prop_kernel(g_hbm, src_hbm, dst_hbm, out_hbm,
                    src_v, dst_v, rows_v, zero_v, acc_sh, g_sh, sem_g, sem_s):
        cid = lax.axis_index("c")
        sid = lax.axis_index("s")
        wid = sid * NC + cid

        pltpu.sync_copy(src_hbm.at[wid], src_v)
        pltpu.sync_copy(dst_hbm.at[wid], dst_v)

        def fill(i, carry):
            zero_v[i, :] = jnp.zeros((f,), jnp.float32)
            return carry
        lax.fori_loop(0, C, fill, 0)

        base = sid * npt
        pltpu.sync_copy(g_hbm.at[pl.ds(base, npt)], g_sh.at[pl.ds(base, npt)])
        for off in range(0, npt, C):
            w = min(C, npt - off)
            pltpu.sync_copy(zero_v.at[pl.ds(0, w)], acc_sh.at[pl.ds(base + off, w)])
        plsc.subcore_barrier()

        D = NBUF - 1
        for d in range(D):
            pltpu.async_copy(g_sh.at[src_v.at[d]], rows_v.at[d], sem_g.at[d])

        def chunk(j, carry):
            b = j % NBUF
            pltpu.make_async_copy(g_sh.at[src_v.at[j]], rows_v.at[b],
                                  sem_g.at[b]).wait()
            pltpu.make_async_copy(rows_v.at[b], acc_sh.at[dst_v.at[j]],
                                  sem_s.at[b]).start(add=True)

            @pl.when(j >= 1)
            def _():
                bp = (j - 1) % NBUF
                pltpu.make_async_copy(rows_v.at[bp], acc_sh.at[dst_v.at[j - 1]],
                                      sem_s.at[bp]).wait()

            @pl.when(j + D < chunks)
            def _():
                bn = (j + D) % NBUF
                pltpu.async_copy(g_sh.at[src_v.at[j + D]], rows_v.at[bn],
                                 sem_g.at[bn])
            return carry
        lax.fori_loop(0, chunks, chunk, 0)
        b_last = (chunks - 1) % NBUF
        pltpu.make_async_copy(rows_v.at[b_last], acc_sh.at[dst_v.at[chunks - 1]],
                              sem_s.at[b_last]).wait()
        plsc.subcore_barrier()

        pltpu.sync_copy(acc_sh.at[pl.ds(base, npt)],
                        out_hbm.at[cid, pl.ds(base, npt)])

    return prop_kernel(g, src_t, dst_t)


def _k1_body(x_ref, w1_ref, degp_ref, g1_ref, dinv_ref):
    n = x_ref.shape[0]
    n_acc = degp_ref.shape[1]
    f = w1_ref.shape[1]
    deg = degp_ref[0] + degp_ref[1] + 1.0
    dinv = lax.rsqrt(deg)
    dinv_ref[...] = dinv
    t1 = jnp.dot(x_ref[...], w1_ref[...], preferred_element_type=jnp.float32)
    t1p = jnp.concatenate([t1, jnp.zeros((n_acc - n, f), jnp.float32)], axis=0)
    g1_ref[...] = dinv * t1p


def _k2_body(acc_ref, g_ref, dinv_ref, b1_ref, wm1_ref, bm1_ref, w2_ref, g2_ref):
    s = acc_ref[0] + acc_ref[1] + g_ref[...]
    h1 = jnp.maximum(dinv_ref[...] * s + b1_ref[...], 0.0)
    h2 = jnp.maximum(
        jnp.dot(h1, wm1_ref[...], preferred_element_type=jnp.float32) + bm1_ref[...],
        0.0)
    g2_ref[...] = dinv_ref[...] * jnp.dot(
        h2, w2_ref[...], preferred_element_type=jnp.float32)


def _k3_body(acc_ref, g_ref, dinv_ref, b2_ref, w3_ref, g3_ref):
    s = acc_ref[0] + acc_ref[1] + g_ref[...]
    h3 = jnp.maximum(dinv_ref[...] * s + b2_ref[...], 0.0)
    g3_ref[...] = dinv_ref[...] * jnp.dot(
        h3, w3_ref[...], preferred_element_type=jnp.float32)


def _k4_body(acc_ref, g_ref, dinv_ref, b3_ref, batch_ref, wl_ref, bl_ref, out_ref):
    n, g_out = batch_ref.shape[0], out_ref.shape[0]
    f = g_ref.shape[1]
    s = acc_ref[0] + acc_ref[1] + g_ref[...]
    h = jnp.maximum(dinv_ref[...] * s + b3_ref[...], 0.0)[:n]
    onehot = (batch_ref[...] == lax.broadcasted_iota(jnp.int32, (1, g_out), 1)
              ).astype(jnp.float32)
    aug = jnp.concatenate([h, jnp.ones((n, 1), jnp.float32)], axis=1)
    sums = lax.dot_general(onehot, aug, (((0,), (0,)), ((), ())),
                           preferred_element_type=jnp.float32)
    pooled = sums[:, :f] / jnp.maximum(sums[:, f:f + 1], 1.0)
    out_ref[...] = jnp.dot(pooled, wl_ref[...],
                           preferred_element_type=jnp.float32) + bl_ref[...]


def _tc(body, out_shapes):
    return pl.pallas_call(body, out_shape=out_shapes)


def kernel(x, edge_index, batch, W1, b1, Wm1, bm1, W2, b2, W3, b3, Wl, bl):
    n, f_in = x.shape
    f = W1.shape[1]
    e = edge_index.shape[1]
    num_graphs = 32

    n_acc = ((n + 1 + 8 * NS - 1) // (8 * NS)) * (8 * NS)
    chunks = -(-e // (NW * C))
    if chunks % 2:
        chunks += 1
    ept_p = chunks * C

    pad = NW * ept_p - e
    dummy = jnp.full((pad,), n, jnp.int32)
    src_t = jnp.concatenate([edge_index[0], dummy]).reshape(NW, chunks, C)
    dst_t = jnp.concatenate([edge_index[1], dummy]).reshape(NW, chunks, C)

    degp = _deg_call(dst_t, n_acc, f)

    sd32 = jax.ShapeDtypeStruct((n_acc, f), jnp.float32)
    g1, dinv = _tc(_k1_body, [sd32, sd32])(x, W1, degp)

    acc1 = _prop_call(g1, src_t, dst_t, n_acc)
    g2 = _tc(_k2_body, sd32)(acc1, g1, dinv, b1.reshape(1, f), Wm1,
                             bm1.reshape(1, f), W2)
    acc2 = _prop_call(g2, src_t, dst_t, n_acc)
    g3 = _tc(_k3_body, sd32)(acc2, g2, dinv, b2.reshape(1, f), W3)
    acc3 = _prop_call(g3, src_t, dst_t, n_acc)
    out = _tc(_k4_body, jax.ShapeDtypeStruct((num_graphs, 1), jnp.float32))(
        acc3, g3, dinv, b3.reshape(1, f), batch.reshape(n, 1), Wl,
        bl.reshape(1, 1))
    return out


# speedup vs baseline: 54.5213x; 1.0032x over previous
import functools

import jax
import jax.numpy as jnp
from jax import lax
from jax.experimental import pallas as pl
from jax.experimental.pallas import tpu as pltpu
from jax.experimental.pallas import tpu_sc as plsc

NC = 2
NS = 16
NW = NC * NS
C = 128
NBUF = 8


def _mesh():
    return plsc.VectorSubcoreMesh(
        core_axis_name="c", subcore_axis_name="s", num_cores=NC, num_subcores=NS
    )


_SC_PARAMS = pltpu.CompilerParams(use_tc_tiling_on_sc=False)


def _deg_call(dst_t, n_acc, f):
    chunks = dst_t.shape[1]
    npt = n_acc // NS

    @functools.partial(
        pl.kernel,
        out_type=jax.ShapeDtypeStruct((NC, n_acc, f), jnp.float32),
        mesh=_mesh(),
        compiler_params=_SC_PARAMS,
        scratch_types=[
            pltpu.VMEM((chunks, C), jnp.int32),
            pltpu.VMEM((C, f), jnp.float32),
            pltpu.VMEM((C, f), jnp.float32),
            pltpu.VMEM_SHARED((n_acc, f), jnp.float32),
        ],
    )
    def deg_kernel(dst_hbm, out_hbm, dst_v, ones_v, zero_v, acc_sh):
        cid = lax.axis_index("c")
        sid = lax.axis_index("s")
        wid = sid * NC + cid

        pltpu.sync_copy(dst_hbm.at[wid], dst_v)

        def fill(i, carry):
            ones_v[i, :] = jnp.full((f,), 1.0, jnp.float32)
            zero_v[i, :] = jnp.zeros((f,), jnp.float32)
            return carry
        lax.fori_loop(0, C, fill, 0)

        base = sid * npt
        for off in range(0, npt, C):
            w = min(C, npt - off)
            pltpu.sync_copy(zero_v.at[pl.ds(0, w)], acc_sh.at[pl.ds(base + off, w)])
        plsc.subcore_barrier()

        def chunk(j, carry):
            pltpu.sync_copy(ones_v, acc_sh.at[dst_v.at[j]], add=True)
            return carry
        lax.fori_loop(0, chunks, chunk, 0)
        plsc.subcore_barrier()

        pltpu.sync_copy(acc_sh.at[pl.ds(base, npt)],
                        out_hbm.at[cid, pl.ds(base, npt)])

    return deg_kernel(dst_t)


def _prop_call(g, src_t, dst_t, n_acc):
    f = g.shape[1]
    chunks = src_t.shape[1]
    npt = n_acc // NS

    @functools.partial(
        pl.kernel,
        out_type=jax.ShapeDtypeStruct((NC, n_acc, f), jnp.float32),
        mesh=_mesh(),
        compiler_params=_SC_PARAMS,
        scratch_types=[
            pltpu.VMEM((chunks, C), jnp.int32),
            pltpu.VMEM((chunks, C), jnp.int32),
            pltpu.VMEM((NBUF, C, f), jnp.float32),
            pltpu.VMEM((C, f), jnp.float32),
            pltpu.VMEM_SHARED((n_acc, f), jnp.float32),
            pltpu.VMEM_SHARED((n_acc, f), jnp.float32),
            pltpu.SemaphoreType.DMA((NBUF,)),
            pltpu.SemaphoreType.DMA((NBUF,)),
        ],
    )
    def prop_kernel(g_hbm, src_hbm, dst_hbm, out_hbm,
                    src_v, dst_v, rows_v, zero_v, acc_sh, g_sh, sem_g, sem_s):
        cid = lax.axis_index("c")
        sid = lax.axis_index("s")
        wid = sid * NC + cid

        pltpu.sync_copy(src_hbm.at[wid], src_v)
        pltpu.sync_copy(dst_hbm.at[wid], dst_v)

        def fill(i, carry):
            zero_v[i, :] = jnp.zeros((f,), jnp.float32)
            return carry
        lax.fori_loop(0, C, fill, 0)

        base = sid * npt
        pltpu.sync_copy(g_hbm.at[pl.ds(base, npt)], g_sh.at[pl.ds(base, npt)])
        for off in range(0, npt, C):
            w = min(C, npt - off)
            pltpu.sync_copy(zero_v.at[pl.ds(0, w)], acc_sh.at[pl.ds(base + off, w)])
        plsc.subcore_barrier()

        D = 4
        for d in range(D):
            pltpu.async_copy(g_sh.at[src_v.at[d]], rows_v.at[d], sem_g.at[d])

        def chunk(j, carry):
            b = j % NBUF
            pltpu.make_async_copy(g_sh.at[src_v.at[j]], rows_v.at[b],
                                  sem_g.at[b]).wait()
            pltpu.make_async_copy(rows_v.at[b], acc_sh.at[dst_v.at[j]],
                                  sem_s.at[b]).start(add=True)
            bn = (j + D) % NBUF

            @pl.when(j + D >= NBUF)
            def _():
                pltpu.make_async_copy(rows_v.at[bn],
                                      acc_sh.at[dst_v.at[j + D - NBUF]],
                                      sem_s.at[bn]).wait()

            @pl.when(j + D < chunks)
            def _():
                pltpu.async_copy(g_sh.at[src_v.at[j + D]], rows_v.at[bn],
                                 sem_g.at[bn])
            return carry
        lax.fori_loop(0, chunks, chunk, 0)
        for t in range(chunks - (NBUF - D), chunks):
            bt = t % NBUF
            pltpu.make_async_copy(rows_v.at[bt], acc_sh.at[dst_v.at[t]],
                                  sem_s.at[bt]).wait()
        plsc.subcore_barrier()

        pltpu.sync_copy(acc_sh.at[pl.ds(base, npt)],
                        out_hbm.at[cid, pl.ds(base, npt)])

    return prop_kernel(g, src_t, dst_t)


def _k0_body(x_ref, w1_ref, t1_ref):
    n = x_ref.shape[0]
    n_acc, f = t1_ref.shape
    t1 = jnp.dot(x_ref[...], w1_ref[...], preferred_element_type=jnp.float32)
    t1_ref[...] = jnp.concatenate(
        [t1, jnp.zeros((n_acc - n, f), jnp.float32)], axis=0)


def _k1_body(t1_ref, degp_ref, g1_ref, dinv_ref):
    deg = degp_ref[0] + degp_ref[1] + 1.0
    dinv = lax.rsqrt(deg)
    dinv_ref[...] = dinv
    g1_ref[...] = dinv * t1_ref[...]


def _k2_body(acc_ref, g_ref, dinv_ref, b1_ref, wm1_ref, bm1_ref, w2_ref, g2_ref):
    s = acc_ref[0] + acc_ref[1] + g_ref[...]
    h1 = jnp.maximum(dinv_ref[...] * s + b1_ref[...], 0.0)
    h2 = jnp.maximum(
        jnp.dot(h1, wm1_ref[...], preferred_element_type=jnp.float32) + bm1_ref[...],
        0.0)
    g2_ref[...] = dinv_ref[...] * jnp.dot(
        h2, w2_ref[...], preferred_element_type=jnp.float32)


def _k3_body(acc_ref, g_ref, dinv_ref, b2_ref, w3_ref, g3_ref):
    s = acc_ref[0] + acc_ref[1] + g_ref[...]
    h3 = jnp.maximum(dinv_ref[...] * s + b2_ref[...], 0.0)
    g3_ref[...] = dinv_ref[...] * jnp.dot(
        h3, w3_ref[...], preferred_element_type=jnp.float32)


def _k4_body(acc_ref, g_ref, dinv_ref, b3_ref, batch_ref, wl_ref, bl_ref, out_ref):
    n, g_out = batch_ref.shape[0], out_ref.shape[0]
    f = g_ref.shape[1]
    s = acc_ref[0] + acc_ref[1] + g_ref[...]
    h = jnp.maximum(dinv_ref[...] * s + b3_ref[...], 0.0)[:n]
    onehot = (batch_ref[...] == lax.broadcasted_iota(jnp.int32, (1, g_out), 1)
              ).astype(jnp.float32)
    aug = jnp.concatenate([h, jnp.ones((n, 1), jnp.float32)], axis=1)
    sums = lax.dot_general(onehot, aug, (((0,), (0,)), ((), ())),
                           preferred_element_type=jnp.float32)
    pooled = sums[:, :f] / jnp.maximum(sums[:, f:f + 1], 1.0)
    out_ref[...] = jnp.dot(pooled, wl_ref[...],
                           preferred_element_type=jnp.float32) + bl_ref[...]


def _tc(body, out_shapes):
    return pl.pallas_call(body, out_shape=out_shapes)


def kernel(x, edge_index, batch, W1, b1, Wm1, bm1, W2, b2, W3, b3, Wl, bl):
    n, f_in = x.shape
    f = W1.shape[1]
    e = edge_index.shape[1]
    num_graphs = 32

    n_acc = ((n + 1 + 8 * NS - 1) // (8 * NS)) * (8 * NS)
    chunks = -(-e // (NW * C))
    if chunks % 2:
        chunks += 1
    ept_p = chunks * C

    pad = NW * ept_p - e
    dummy = jnp.full((pad,), n, jnp.int32)
    src_t = jnp.concatenate([edge_index[0], dummy]).reshape(NW, chunks, C)
    dst_t = jnp.concatenate([edge_index[1], dummy]).reshape(NW, chunks, C)

    sd32 = jax.ShapeDtypeStruct((n_acc, f), jnp.float32)
    t1 = _tc(_k0_body, sd32)(x, W1)
    degp = _deg_call(dst_t, n_acc, f)
    g1, dinv = _tc(_k1_body, [sd32, sd32])(t1, degp)

    acc1 = _prop_call(g1, src_t, dst_t, n_acc)
    g2 = _tc(_k2_body, sd32)(acc1, g1, dinv, b1.reshape(1, f), Wm1,
                             bm1.reshape(1, f), W2)
    acc2 = _prop_call(g2, src_t, dst_t, n_acc)
    g3 = _tc(_k3_body, sd32)(acc2, g2, dinv, b2.reshape(1, f), W3)
    acc3 = _prop_call(g3, src_t, dst_t, n_acc)
    out = _tc(_k4_body, jax.ShapeDtypeStruct((num_graphs, 1), jnp.float32))(
        acc3, g3, dinv, b3.reshape(1, f), batch.reshape(n, 1), Wl,
        bl.reshape(1, 1))
    return out


# speedup vs baseline: 62.6286x; 1.1487x over previous
import functools

import jax
import jax.numpy as jnp
from jax import lax
from jax.experimental import pallas as pl
from jax.experimental.pallas import tpu as pltpu
from jax.experimental.pallas import tpu_sc as plsc

NC = 2
NS = 16
NW = NC * NS
C = 128
NBUF = 8


def _mesh():
    return plsc.VectorSubcoreMesh(
        core_axis_name="c", subcore_axis_name="s", num_cores=NC, num_subcores=NS
    )


_SC_PARAMS = pltpu.CompilerParams(use_tc_tiling_on_sc=False)


def _my_rows(wid, rpt, rem):
    return jnp.where(wid < rem, rpt + 1, rpt)


def _copy_idx(ei_hbm, k, wid, v, rpt, rem):
    pltpu.sync_copy(ei_hbm.at[k, pl.ds(wid * rpt, rpt)], v.at[pl.ds(0, rpt)])

    @pl.when(wid < rem)
    def _():
        pltpu.sync_copy(ei_hbm.at[k, pl.ds(rpt * NW + wid, 1)],
                        v.at[pl.ds(rpt, 1)])


def _deg_call(ei3, n_acc, f, rpt, rem):
    npt = n_acc // NS

    @functools.partial(
        pl.kernel,
        out_type=jax.ShapeDtypeStruct((NC, n_acc, f), jnp.float32),
        mesh=_mesh(),
        compiler_params=_SC_PARAMS,
        scratch_types=[
            pltpu.VMEM((rpt + 1, C), jnp.int32),
            pltpu.VMEM((C, f), jnp.float32),
            pltpu.VMEM((C, f), jnp.float32),
            pltpu.VMEM_SHARED((n_acc, f), jnp.float32),
        ],
    )
    def deg_kernel(ei_hbm, out_hbm, dst_v, ones_v, zero_v, acc_sh):
        cid = lax.axis_index("c")
        sid = lax.axis_index("s")
        wid = sid * NC + cid
        nrows = _my_rows(wid, rpt, rem)

        _copy_idx(ei_hbm, 1, wid, dst_v, rpt, rem)

        def fill(i, carry):
            ones_v[i, :] = jnp.full((f,), 1.0, jnp.float32)
            zero_v[i, :] = jnp.zeros((f,), jnp.float32)
            return carry
        lax.fori_loop(0, C, fill, 0)

        base = sid * npt
        for off in range(0, npt, C):
            w = min(C, npt - off)
            pltpu.sync_copy(zero_v.at[pl.ds(0, w)], acc_sh.at[pl.ds(base + off, w)])
        plsc.subcore_barrier()

        def chunk(j, carry):
            pltpu.sync_copy(ones_v, acc_sh.at[dst_v.at[j]], add=True)
            return carry
        lax.fori_loop(0, nrows, chunk, 0)
        plsc.subcore_barrier()

        pltpu.sync_copy(acc_sh.at[pl.ds(base, npt)],
                        out_hbm.at[cid, pl.ds(base, npt)])

    return deg_kernel(ei3)


def _prop_call(g, ei3, n_acc, rpt, rem):
    f = g.shape[1]
    npt = n_acc // NS

    @functools.partial(
        pl.kernel,
        out_type=jax.ShapeDtypeStruct((NC, n_acc, f), jnp.float32),
        mesh=_mesh(),
        compiler_params=_SC_PARAMS,
        scratch_types=[
            pltpu.VMEM((rpt + 1, C), jnp.int32),
            pltpu.VMEM((rpt + 1, C), jnp.int32),
            pltpu.VMEM((NBUF, C, f), jnp.float32),
            pltpu.VMEM((C, f), jnp.float32),
            pltpu.VMEM_SHARED((n_acc, f), jnp.float32),
            pltpu.VMEM_SHARED((n_acc, f), jnp.float32),
            pltpu.SemaphoreType.DMA((NBUF,)),
            pltpu.SemaphoreType.DMA((NBUF,)),
        ],
    )
    def prop_kernel(g_hbm, ei_hbm, out_hbm,
                    src_v, dst_v, rows_v, zero_v, acc_sh, g_sh, sem_g, sem_s):
        cid = lax.axis_index("c")
        sid = lax.axis_index("s")
        wid = sid * NC + cid
        nrows = _my_rows(wid, rpt, rem)

        _copy_idx(ei_hbm, 0, wid, src_v, rpt, rem)
        _copy_idx(ei_hbm, 1, wid, dst_v, rpt, rem)

        def fill(i, carry):
            zero_v[i, :] = jnp.zeros((f,), jnp.float32)
            return carry
        lax.fori_loop(0, C, fill, 0)

        base = sid * npt
        pltpu.sync_copy(g_hbm.at[pl.ds(base, npt)], g_sh.at[pl.ds(base, npt)])
        for off in range(0, npt, C):
            w = min(C, npt - off)
            pltpu.sync_copy(zero_v.at[pl.ds(0, w)], acc_sh.at[pl.ds(base + off, w)])
        plsc.subcore_barrier()

        D = 4
        for d in range(D):
            pltpu.async_copy(g_sh.at[src_v.at[d]], rows_v.at[d], sem_g.at[d])

        def chunk(j, carry):
            b = j % NBUF
            pltpu.make_async_copy(g_sh.at[src_v.at[j]], rows_v.at[b],
                                  sem_g.at[b]).wait()
            pltpu.make_async_copy(rows_v.at[b], acc_sh.at[dst_v.at[j]],
                                  sem_s.at[b]).start(add=True)
            bn = (j + D) % NBUF

            @pl.when(j + D >= NBUF)
            def _():
                pltpu.make_async_copy(rows_v.at[bn],
                                      acc_sh.at[dst_v.at[j + D - NBUF]],
                                      sem_s.at[bn]).wait()

            @pl.when(j + D < nrows)
            def _():
                pltpu.async_copy(g_sh.at[src_v.at[j + D]], rows_v.at[bn],
                                 sem_g.at[bn])
            return carry
        lax.fori_loop(0, nrows, chunk, 0)

        def drain(t, carry):
            bt = t % NBUF
            pltpu.make_async_copy(rows_v.at[bt], acc_sh.at[dst_v.at[t]],
                                  sem_s.at[bt]).wait()
            return carry
        lax.fori_loop(nrows - (NBUF - D), nrows, drain, 0)
        plsc.subcore_barrier()

        pltpu.sync_copy(acc_sh.at[pl.ds(base, npt)],
                        out_hbm.at[cid, pl.ds(base, npt)])

    return prop_kernel(g, ei3)


def _k0_body(x_ref, w1_ref, t1_ref):
    n = x_ref.shape[0]
    n_acc, f = t1_ref.shape
    t1 = jnp.dot(x_ref[...], w1_ref[...], preferred_element_type=jnp.float32)
    t1_ref[...] = jnp.concatenate(
        [t1, jnp.zeros((n_acc - n, f), jnp.float32)], axis=0)


def _k1_body(t1_ref, degp_ref, g1_ref, dinv_ref):
    deg = degp_ref[0] + degp_ref[1] + 1.0
    dinv = lax.rsqrt(deg)
    dinv_ref[...] = dinv
    g1_ref[...] = dinv * t1_ref[...]


def _k2_body(acc_ref, g_ref, dinv_ref, b1_ref, wm1_ref, bm1_ref, w2_ref, g2_ref):
    s = acc_ref[0] + acc_ref[1] + g_ref[...]
    h1 = jnp.maximum(dinv_ref[...] * s + b1_ref[...], 0.0)
    h2 = jnp.maximum(
        jnp.dot(h1, wm1_ref[...], preferred_element_type=jnp.float32) + bm1_ref[...],
        0.0)
    g2_ref[...] = dinv_ref[...] * jnp.dot(
        h2, w2_ref[...], preferred_element_type=jnp.float32)


def _k3_body(acc_ref, g_ref, dinv_ref, b2_ref, w3_ref, g3_ref):
    s = acc_ref[0] + acc_ref[1] + g_ref[...]
    h3 = jnp.maximum(dinv_ref[...] * s + b2_ref[...], 0.0)
    g3_ref[...] = dinv_ref[...] * jnp.dot(
        h3, w3_ref[...], preferred_element_type=jnp.float32)


def _k4_body(acc_ref, g_ref, dinv_ref, b3_ref, batch_ref, wl_ref, bl_ref, out_ref):
    n, g_out = batch_ref.shape[0], out_ref.shape[0]
    f = g_ref.shape[1]
    s = acc_ref[0] + acc_ref[1] + g_ref[...]
    h = jnp.maximum(dinv_ref[...] * s + b3_ref[...], 0.0)[:n]
    onehot = (batch_ref[...] == lax.broadcasted_iota(jnp.int32, (1, g_out), 1)
              ).astype(jnp.float32)
    aug = jnp.concatenate([h, jnp.ones((n, 1), jnp.float32)], axis=1)
    sums = lax.dot_general(onehot, aug, (((0,), (0,)), ((), ())),
                           preferred_element_type=jnp.float32)
    pooled = sums[:, :f] / jnp.maximum(sums[:, f:f + 1], 1.0)
    out_ref[...] = jnp.dot(pooled, wl_ref[...],
                           preferred_element_type=jnp.float32) + bl_ref[...]


def _tc(body, out_shapes):
    return pl.pallas_call(body, out_shape=out_shapes)


def kernel(x, edge_index, batch, W1, b1, Wm1, bm1, W2, b2, W3, b3, Wl, bl):
    n, f_in = x.shape
    f = W1.shape[1]
    e = edge_index.shape[1]
    num_graphs = 32

    n_acc = ((n + 1 + 8 * NS - 1) // (8 * NS)) * (8 * NS)

    if e % C:
        pad = C - e % C
        edge_index = jnp.pad(edge_index, ((0, 0), (0, pad)), constant_values=n)
    r = edge_index.shape[1] // C
    rpt, rem = r // NW, r % NW
    ei3 = edge_index.reshape(2, r, C)

    sd32 = jax.ShapeDtypeStruct((n_acc, f), jnp.float32)
    t1 = _tc(_k0_body, sd32)(x, W1)
    degp = _deg_call(ei3, n_acc, f, rpt, rem)
    g1, dinv = _tc(_k1_body, [sd32, sd32])(t1, degp)

    acc1 = _prop_call(g1, ei3, n_acc, rpt, rem)
    g2 = _tc(_k2_body, sd32)(acc1, g1, dinv, b1.reshape(1, f), Wm1,
                             bm1.reshape(1, f), W2)
    acc2 = _prop_call(g2, ei3, n_acc, rpt, rem)
    g3 = _tc(_k3_body, sd32)(acc2, g2, dinv, b2.reshape(1, f), W3)
    acc3 = _prop_call(g3, ei3, n_acc, rpt, rem)
    out = _tc(_k4_body, jax.ShapeDtypeStruct((num_graphs, 1), jnp.float32))(
        acc3, g3, dinv, b3.reshape(1, f), batch.reshape(n, 1), Wl,
        bl.reshape(1, 1))
    return out


# speedup vs baseline: 101.5123x; 1.6209x over previous
import functools

import jax
import jax.numpy as jnp
from jax import lax
from jax.experimental import pallas as pl
from jax.experimental.pallas import tpu as pltpu
from jax.experimental.pallas import tpu_sc as plsc

NC = 2
NS = 16
NW = NC * NS
C = 128
NBUF = 8


def _mesh():
    return plsc.VectorSubcoreMesh(
        core_axis_name="c", subcore_axis_name="s", num_cores=NC, num_subcores=NS
    )


_SC_PARAMS = pltpu.CompilerParams(use_tc_tiling_on_sc=False)


def _my_rows(wid, rpt, rem):
    return jnp.where(wid < rem, rpt + 1, rpt)


def _copy_idx(ei_hbm, k, wid, v, rpt, rem):
    pltpu.sync_copy(ei_hbm.at[k, pl.ds(wid * rpt, rpt)], v.at[pl.ds(0, rpt)])

    @pl.when(wid < rem)
    def _():
        pltpu.sync_copy(ei_hbm.at[k, pl.ds(rpt * NW + wid, 1)],
                        v.at[pl.ds(rpt, 1)])


def _deg_call(ei3, n_acc, f, rpt, rem):
    npt = n_acc // NS

    @functools.partial(
        pl.kernel,
        out_type=jax.ShapeDtypeStruct((NC, n_acc, f), jnp.float32),
        mesh=_mesh(),
        compiler_params=_SC_PARAMS,
        scratch_types=[
            pltpu.VMEM((rpt + 1, C), jnp.int32),
            pltpu.VMEM((C, f), jnp.float32),
            pltpu.VMEM((C, f), jnp.float32),
            pltpu.VMEM_SHARED((n_acc, f), jnp.float32),
        ],
    )
    def deg_kernel(ei_hbm, out_hbm, dst_v, ones_v, zero_v, acc_sh):
        cid = lax.axis_index("c")
        sid = lax.axis_index("s")
        wid = sid * NC + cid
        nrows = _my_rows(wid, rpt, rem)

        _copy_idx(ei_hbm, 1, wid, dst_v, rpt, rem)

        def fill(i, carry):
            ones_v[i, :] = jnp.full((f,), 1.0, jnp.float32)
            zero_v[i, :] = jnp.zeros((f,), jnp.float32)
            return carry
        lax.fori_loop(0, C, fill, 0)

        base = sid * npt
        for off in range(0, npt, C):
            w = min(C, npt - off)
            pltpu.sync_copy(zero_v.at[pl.ds(0, w)], acc_sh.at[pl.ds(base + off, w)])
        plsc.subcore_barrier()

        def chunk(j, carry):
            pltpu.sync_copy(ones_v, acc_sh.at[dst_v.at[j]], add=True)
            return carry
        lax.fori_loop(0, nrows, chunk, 0)
        plsc.subcore_barrier()

        pltpu.sync_copy(acc_sh.at[pl.ds(base, npt)],
                        out_hbm.at[cid, pl.ds(base, npt)])

    return deg_kernel(ei3)


def _prop_call(g, ei3, n_acc, rpt, rem):
    f = g.shape[1]
    npt = n_acc // NS

    @functools.partial(
        pl.kernel,
        out_type=jax.ShapeDtypeStruct((NC, n_acc, f), jnp.float32),
        mesh=_mesh(),
        compiler_params=_SC_PARAMS,
        scratch_types=[
            pltpu.VMEM((rpt + 1, C), jnp.int32),
            pltpu.VMEM((rpt + 1, C), jnp.int32),
            pltpu.VMEM((NBUF, C, f), jnp.float32),
            pltpu.VMEM((C, f), jnp.float32),
            pltpu.VMEM_SHARED((n_acc, f), jnp.float32),
            pltpu.VMEM_SHARED((n_acc, f), jnp.float32),
            pltpu.SemaphoreType.DMA((NBUF,)),
            pltpu.SemaphoreType.DMA((NBUF,)),
        ],
    )
    def prop_kernel(g_hbm, ei_hbm, out_hbm,
                    src_v, dst_v, rows_v, zero_v, acc_sh, g_sh, sem_g, sem_s):
        cid = lax.axis_index("c")
        sid = lax.axis_index("s")
        wid = sid * NC + cid
        nrows = _my_rows(wid, rpt, rem)

        _copy_idx(ei_hbm, 0, wid, src_v, rpt, rem)
        _copy_idx(ei_hbm, 1, wid, dst_v, rpt, rem)

        def fill(i, carry):
            zero_v[i, :] = jnp.zeros((f,), jnp.float32)
            return carry
        lax.fori_loop(0, C, fill, 0)

        base = sid * npt
        pltpu.sync_copy(g_hbm.at[pl.ds(base, npt)], g_sh.at[pl.ds(base, npt)])
        for off in range(0, npt, C):
            w = min(C, npt - off)
            pltpu.sync_copy(zero_v.at[pl.ds(0, w)], acc_sh.at[pl.ds(base + off, w)])
        plsc.subcore_barrier()

        D = 4
        for d in range(D):
            pltpu.async_copy(g_sh.at[src_v.at[d]], rows_v.at[d], sem_g.at[d])

        def chunk(j, carry):
            b = j % NBUF
            pltpu.make_async_copy(g_sh.at[src_v.at[j]], rows_v.at[b],
                                  sem_g.at[b]).wait()
            pltpu.make_async_copy(rows_v.at[b], acc_sh.at[dst_v.at[j]],
                                  sem_s.at[b]).start(add=True)
            bn = (j + D) % NBUF

            @pl.when(j + D >= NBUF)
            def _():
                pltpu.make_async_copy(rows_v.at[bn],
                                      acc_sh.at[dst_v.at[j + D - NBUF]],
                                      sem_s.at[bn]).wait()

            @pl.when(j + D < nrows)
            def _():
                pltpu.async_copy(g_sh.at[src_v.at[j + D]], rows_v.at[bn],
                                 sem_g.at[bn])
            return carry
        lax.fori_loop(0, nrows, chunk, 0)

        def drain(t, carry):
            bt = t % NBUF
            pltpu.make_async_copy(rows_v.at[bt], acc_sh.at[dst_v.at[t]],
                                  sem_s.at[bt]).wait()
            return carry
        lax.fori_loop(nrows - (NBUF - D), nrows, drain, 0)
        plsc.subcore_barrier()

        pltpu.sync_copy(acc_sh.at[pl.ds(base, npt)],
                        out_hbm.at[cid, pl.ds(base, npt)])

    return prop_kernel(g, ei3)


def _kron8(w_ref):
    w = w_ref[...]
    fi, fo = w.shape
    blocks = []
    for a in range(8):
        row = w
        if a:
            row = jnp.concatenate(
                [jnp.zeros((fi, a * fo), jnp.float32), row], axis=1)
        if a < 7:
            row = jnp.concatenate(
                [row, jnp.zeros((fi, (7 - a) * fo), jnp.float32)], axis=1)
        blocks.append(row)
    return jnp.concatenate(blocks, axis=0)


def _tile8(b_ref):
    return jnp.tile(b_ref[...], (1, 8))


def _k0_body(xg_ref, w1_ref, t1_ref):
    npr = xg_ref.shape[0]
    np_, lanes = t1_ref.shape
    t1 = jnp.dot(xg_ref[...], _kron8(w1_ref),
                 preferred_element_type=jnp.float32)
    t1_ref[...] = jnp.concatenate(
        [t1, jnp.zeros((np_ - npr, lanes), jnp.float32)], axis=0)


def _k1_body(t1_ref, degp_ref, g1_ref, dinv_ref):
    deg = degp_ref[0] + degp_ref[1] + 1.0
    dinv = lax.rsqrt(deg)
    dinv_ref[...] = dinv
    g1_ref[...] = dinv * t1_ref[...]


def _k2_body(acc_ref, g_ref, dinv_ref, b1_ref, wm1_ref, bm1_ref, w2_ref, g2_ref):
    s = acc_ref[0] + acc_ref[1] + g_ref[...]
    h1 = jnp.maximum(dinv_ref[...] * s + _tile8(b1_ref), 0.0)
    h2 = jnp.maximum(
        jnp.dot(h1, _kron8(wm1_ref), preferred_element_type=jnp.float32)
        + _tile8(bm1_ref), 0.0)
    g2_ref[...] = dinv_ref[...] * jnp.dot(
        h2, _kron8(w2_ref), preferred_element_type=jnp.float32)


def _k3_body(acc_ref, g_ref, dinv_ref, b2_ref, w3_ref, g3_ref):
    s = acc_ref[0] + acc_ref[1] + g_ref[...]
    h3 = jnp.maximum(dinv_ref[...] * s + _tile8(b2_ref), 0.0)
    g3_ref[...] = dinv_ref[...] * jnp.dot(
        h3, _kron8(w3_ref), preferred_element_type=jnp.float32)


def _k4_body(acc_ref, g_ref, dinv_ref, b3_ref, batchp_ref, wl_ref, bl_ref,
             out_ref):
    g_out = out_ref.shape[0]
    lanes = g_ref.shape[1]
    f = lanes // 8
    ge = 8 * g_out
    s = acc_ref[0] + acc_ref[1] + g_ref[...]
    hp = jnp.maximum(dinv_ref[...] * s + _tile8(b3_ref), 0.0)
    bp = batchp_ref[...]
    gid = lax.broadcasted_iota(jnp.int32, (1, ge), 1) // 8
    ppk = (jnp.tile(bp, (1, g_out)) == gid).astype(jnp.float32)
    s2 = lax.dot_general(ppk, hp, (((0,), (0,)), ((), ())),
                         preferred_element_type=jnp.float32)
    keep = (lax.broadcasted_iota(jnp.int32, (ge, lanes), 0) % 8
            == lax.broadcasted_iota(jnp.int32, (ge, lanes), 1) // f)
    d2 = jnp.where(keep, s2, 0.0)
    nsum = (lax.broadcasted_iota(jnp.int32, (g_out, ge), 1) // 8
            == lax.broadcasted_iota(jnp.int32, (g_out, ge), 0)
            ).astype(jnp.float32)
    msum = (lax.broadcasted_iota(jnp.int32, (lanes, f), 0) % f
            == lax.broadcasted_iota(jnp.int32, (lanes, f), 1)
            ).astype(jnp.float32)
    sums = jnp.dot(jnp.dot(nsum, d2, preferred_element_type=jnp.float32),
                   msum, preferred_element_type=jnp.float32)
    ones_col = jnp.ones((ppk.shape[0], 1), jnp.float32)
    cnt = jnp.dot(nsum,
                  lax.dot_general(ppk, ones_col, (((0,), (0,)), ((), ())),
                                  preferred_element_type=jnp.float32),
                  preferred_element_type=jnp.float32)
    pooled = sums / jnp.maximum(cnt, 1.0)
    out_ref[...] = jnp.dot(pooled, wl_ref[...],
                           preferred_element_type=jnp.float32) + bl_ref[...]


def _tc(body, out_shapes):
    return pl.pallas_call(body, out_shape=out_shapes)


def kernel(x, edge_index, batch, W1, b1, Wm1, bm1, W2, b2, W3, b3, Wl, bl):
    n, f_in = x.shape
    f = W1.shape[1]
    e = edge_index.shape[1]
    num_graphs = 32

    n_acc = ((n + 1 + 8 * NS - 1) // (8 * NS)) * (8 * NS)

    if e % C:
        pad = C - e % C
        edge_index = jnp.pad(edge_index, ((0, 0), (0, pad)), constant_values=n)
    r = edge_index.shape[1] // C
    rpt, rem = r // NW, r % NW
    ei3 = edge_index.reshape(2, r, C)

    np_ = n_acc * f // 128
    pk = jax.ShapeDtypeStruct((np_, 128), jnp.float32)
    xg = x.reshape(n // 8, 8 * f_in)
    batchp = jnp.full((n_acc,), -1, jnp.int32).at[:n].set(batch)
    t1 = _tc(_k0_body, pk)(xg, W1)
    degp = _deg_call(ei3, n_acc, f, rpt, rem)
    g1, dinv = _tc(_k1_body, [pk, pk])(t1, degp.reshape(NC, np_, 128))

    acc1 = _prop_call(g1.reshape(n_acc, f), ei3, n_acc, rpt, rem)
    g2 = _tc(_k2_body, pk)(acc1.reshape(NC, np_, 128), g1, dinv,
                           b1.reshape(1, f), Wm1, bm1.reshape(1, f), W2)
    acc2 = _prop_call(g2.reshape(n_acc, f), ei3, n_acc, rpt, rem)
    g3 = _tc(_k3_body, pk)(acc2.reshape(NC, np_, 128), g2, dinv,
                           b2.reshape(1, f), W3)
    acc3 = _prop_call(g3.reshape(n_acc, f), ei3, n_acc, rpt, rem)
    out = _tc(_k4_body, jax.ShapeDtypeStruct((num_graphs, 1), jnp.float32))(
        acc3.reshape(NC, np_, 128), g3, dinv, b3.reshape(1, f),
        batchp.reshape(np_, 8), Wl, bl.reshape(1, 1))
    return out
